# baseline (speedup 1.0000x reference)
import functools
import math

import jax
import jax.numpy as jnp
import numpy as np
from jax import lax
from jax.experimental import pallas as pl
from jax.experimental.pallas import tpu as pltpu
from jax.experimental.pallas import tpu_sc as plsc

_D_MODEL = 768
_MAX_LEN = 2048

_NC = 2
_NS = 16
_NW = _NC * _NS


def _positional_encoding_np(max_len, d_model):
    pos = np.arange(max_len, dtype=np.float32)[:, None]
    div = np.exp(
        np.arange(0, d_model, 2, dtype=np.float32) * (-math.log(10000.0) / d_model)
    )
    pe = np.zeros((max_len, d_model), dtype=np.float32)
    pe[:, 0::2] = np.sin(pos * div)
    pe[:, 1::2] = np.cos(pos * div)
    return pe


_PE = _positional_encoding_np(_MAX_LEN, _D_MODEL)

_BLK = 16
_POS_PER_W = 64
_NQ = _POS_PER_W // _BLK


@functools.partial(jax.jit, static_argnums=(3, 4))
def _embed(x, pe, table, batch, seq_len):
    d = table.shape[1]
    nb = batch * seq_len
    scale = np.float32(math.sqrt(d))
    nvec = d // 16
    half = _BLK // 2

    mesh = plsc.VectorSubcoreMesh(core_axis_name="c", subcore_axis_name="s")

    @functools.partial(
        pl.kernel,
        out_type=jax.ShapeDtypeStruct((nb, d), jnp.float32),
        mesh=mesh,
        scratch_types=[
            [pltpu.VMEM((_BLK,), jnp.int32) for _ in range(_NQ * batch)],
            [pltpu.VMEM((_BLK, d), jnp.float32) for _ in range(2)],
            [pltpu.VMEM((_BLK, d), jnp.float32) for _ in range(2 * batch)],
            [pltpu.SemaphoreType.DMA for _ in range(2)],
            [pltpu.SemaphoreType.DMA for _ in range(2)],
            [pltpu.SemaphoreType.DMA for _ in range(2)],
            pltpu.SemaphoreType.DMA,
        ],
    )
    def body(x_ref, pe_ref, tab_ref, out_ref, idx_v, peb, tok, gsem, osem,
             psem, ixsem):
        wid = lax.axis_index("s") * _NC + lax.axis_index("c")
        p0 = wid * _POS_PER_W

        pec = [None, None]
        pec[0] = pltpu.async_copy(pe_ref.at[pl.ds(p0, _BLK)], peb[0], psem[0])
        pec[1] = pltpu.async_copy(
            pe_ref.at[pl.ds(p0 + _BLK, _BLK)], peb[1], psem[1]
        )
        ixc = []
        for q in range(_NQ):
            for b in range(batch):
                ixc.append(pltpu.async_copy(
                    x_ref.at[b, pl.ds(p0 + q * _BLK, _BLK)],
                    idx_v[q * batch + b], ixsem,
                ))
        for h_ in ixc:
            h_.wait()

        def fire_gathers(q):
            g = q % 2
            return [
                pltpu.async_copy(
                    tab_ref.at[idx_v[q * batch + b]],
                    tok[g * batch + b], gsem[g],
                )
                for b in range(batch)
            ]

        def fire_outs(q):
            g = q % 2
            return [
                pltpu.async_copy(
                    tok[g * batch + b],
                    out_ref.at[pl.ds(b * seq_len + p0 + q * _BLK, _BLK)],
                    osem[g],
                )
                for b in range(batch)
            ]

        def make_half(q):
            g = q % 2
            def row(r, carry):
                for j in range(nvec):
                    sl = pl.ds(j * 16, 16)
                    pv = peb[g][r, sl]
                    for b in range(batch):
                        tok[g * batch + b][r, sl] = (
                            tok[g * batch + b][r, sl] * scale + pv
                        )
                return carry
            return row

        ga = [None, None]
        oc = [None, None]
        ga[0] = fire_gathers(0)
        ga[1] = fire_gathers(1)

        for q in range(_NQ):
            g = q % 2
            for h_ in ga[g]:
                h_.wait()
            pec[g].wait()
            row = make_half(q)
            lax.fori_loop(0, half, row, 0)
            if q >= 1:
                go = 1 - g
                for h_ in oc[go]:
                    h_.wait()
                if q + 1 < _NQ:
                    ga[go] = fire_gathers(q + 1)
            lax.fori_loop(half, _BLK, row, 0)
            oc[g] = fire_outs(q)
            if q + 2 < _NQ:
                pec[g] = pltpu.async_copy(
                    pe_ref.at[pl.ds(p0 + (q + 2) * _BLK, _BLK)], peb[g],
                    psem[g],
                )

        for h_ in oc[(_NQ - 1) % 2]:
            h_.wait()

    return body(x, pe, table)


def kernel(x, table):
    batch, seq_len = x.shape
    d = table.shape[1]
    pe = jnp.asarray(_PE[:seq_len])
    out = _embed(x.astype(jnp.int32), pe, table, batch, seq_len)
    return out.reshape(batch, seq_len, d)


# speedup vs baseline: 1.0726x; 1.0726x over previous
import functools
import math

import jax
import jax.numpy as jnp
import numpy as np
from jax import lax
from jax.experimental import pallas as pl
from jax.experimental.pallas import tpu as pltpu
from jax.experimental.pallas import tpu_sc as plsc

_D_MODEL = 768
_MAX_LEN = 2048

_NC = 2
_NS = 16
_NW = _NC * _NS


def _positional_encoding_np(max_len, d_model):
    pos = np.arange(max_len, dtype=np.float32)[:, None]
    div = np.exp(
        np.arange(0, d_model, 2, dtype=np.float32) * (-math.log(10000.0) / d_model)
    )
    pe = np.zeros((max_len, d_model), dtype=np.float32)
    pe[:, 0::2] = np.sin(pos * div)
    pe[:, 1::2] = np.cos(pos * div)
    return pe


_PE = _positional_encoding_np(_MAX_LEN, _D_MODEL)

_CHUNK = 32
_POS_PER_W = 64
_NBUF = 3


@functools.partial(jax.jit, static_argnums=(3, 4))
def _embed(x, pe, table, batch, seq_len):
    d = table.shape[1]
    nb = batch * seq_len
    halves = _POS_PER_W // _CHUNK
    nchunk = batch * halves
    scale = np.float32(math.sqrt(d))
    nvec = d // 16

    mesh = plsc.VectorSubcoreMesh(core_axis_name="c", subcore_axis_name="s")

    @functools.partial(
        pl.kernel,
        out_type=jax.ShapeDtypeStruct((nb, d), jnp.float32),
        mesh=mesh,
        scratch_types=[
            [pltpu.VMEM((_CHUNK,), jnp.int32) for _ in range(batch * halves)],
            pltpu.VMEM((_CHUNK, d), jnp.float32),
            pltpu.VMEM((_CHUNK, d), jnp.float32),
            pltpu.VMEM((_CHUNK, d), jnp.float32),
            pltpu.VMEM((_CHUNK, d), jnp.float32),
            pltpu.VMEM((_CHUNK, d), jnp.float32),
            pltpu.SemaphoreType.DMA,
            pltpu.SemaphoreType.DMA,
            pltpu.SemaphoreType.DMA,
            pltpu.SemaphoreType.DMA,
            pltpu.SemaphoreType.DMA,
            pltpu.SemaphoreType.DMA,
            pltpu.SemaphoreType.DMA,
            pltpu.SemaphoreType.DMA,
        ],
    )
    def body(x_ref, pe_ref, tab_ref, out_ref, idx_v, pe0, pe1, t0, t1, t2,
             gs0, gs1, gs2, os0, os1, os2, psem, ixsem):
        toks = (t0, t1, t2)
        pes = (pe0, pe1)
        gsem = (gs0, gs1, gs2)
        osem = (os0, os1, os2)
        wid = lax.axis_index("s") * _NC + lax.axis_index("c")
        p0 = wid * _POS_PER_W
        pec0 = pltpu.async_copy(pe_ref.at[pl.ds(p0, _CHUNK)], pe0, psem)
        pec1 = pltpu.async_copy(pe_ref.at[pl.ds(p0 + _CHUNK, _CHUNK)], pe1, psem)
        ixc = []
        for c in range(nchunk):
            h, b = divmod(c, batch)
            ixc.append(pltpu.async_copy(
                x_ref.at[b, pl.ds(p0 + h * _CHUNK, _CHUNK)], idx_v[c], ixsem
            ))
        for h_ in ixc:
            h_.wait()

        def gather(c, buf):
            return pltpu.async_copy(tab_ref.at[idx_v[c]], toks[buf], gsem[buf])

        ga = [None] * _NBUF
        oc = [None] * _NBUF
        ga[0] = gather(0, 0)
        ga[1] = gather(1, 1)
        pec0.wait()
        pec1.wait()

        def run_rows(buf, peh):
            @plsc.parallel_loop(0, _CHUNK, unroll=2)
            def _rows(r):
                for j in range(nvec):
                    sl = pl.ds(j * 16, 16)
                    buf[r, sl] = buf[r, sl] * scale + peh[r, sl]

        for c in range(nchunk):
            a = c % _NBUF
            h, b = divmod(c, batch)
            ga[a].wait()
            nxt = c + _NBUF - 1
            if nxt < nchunk:
                nb_ = nxt % _NBUF
                if oc[nb_] is not None:
                    oc[nb_].wait()
                ga[nb_] = gather(nxt, nb_)
            run_rows(toks[a], pes[h])
            oc[a] = pltpu.async_copy(
                toks[a],
                out_ref.at[pl.ds(b * seq_len + p0 + h * _CHUNK, _CHUNK)],
                osem[a],
            )

        for k in range(_NBUF):
            oc[(nchunk - _NBUF + k) % _NBUF].wait()

    return body(x, pe, table)


def kernel(x, table):
    batch, seq_len = x.shape
    d = table.shape[1]
    pe = jnp.asarray(_PE[:seq_len])
    out = _embed(x.astype(jnp.int32), pe, table, batch, seq_len)
    return out.reshape(batch, seq_len, d)


# speedup vs baseline: 1.1167x; 1.0411x over previous
import functools
import math

import jax
import jax.numpy as jnp
import numpy as np
from jax import lax
from jax.experimental import pallas as pl
from jax.experimental.pallas import tpu as pltpu
from jax.experimental.pallas import tpu_sc as plsc

_D_MODEL = 768
_MAX_LEN = 2048

_NC = 2
_NS = 16
_NW = _NC * _NS


def _positional_encoding_np(max_len, d_model):
    pos = np.arange(max_len, dtype=np.float32)[:, None]
    div = np.exp(
        np.arange(0, d_model, 2, dtype=np.float32) * (-math.log(10000.0) / d_model)
    )
    pe = np.zeros((max_len, d_model), dtype=np.float32)
    pe[:, 0::2] = np.sin(pos * div)
    pe[:, 1::2] = np.cos(pos * div)
    return pe


def _shuffle_for_unpack(pe):
    n, d = pe.shape
    return pe.reshape(n, d // 32, 2, 16).transpose(0, 1, 3, 2).reshape(n, d)


_PE = _positional_encoding_np(_MAX_LEN, _D_MODEL)
_PE_SHUF = _shuffle_for_unpack(_PE)

_CHUNK = 32
_POS_PER_W = 64
_NBUF = 4


@functools.partial(jax.jit, static_argnums=(3, 4))
def _embed(x, pe, table, batch, seq_len):
    d = table.shape[1]
    nb = batch * seq_len
    halves = _POS_PER_W // _CHUNK
    nchunk = batch * halves
    npair = nchunk // 2
    scale = np.float32(math.sqrt(d))
    nv2 = d // 32
    half_rows = _CHUNK // 2

    mesh = plsc.VectorSubcoreMesh(core_axis_name="c", subcore_axis_name="s")

    @functools.partial(
        pl.kernel,
        out_type=jax.ShapeDtypeStruct((nb, d), jnp.float32),
        mesh=mesh,
        scratch_types=[
            [pltpu.VMEM((_CHUNK,), jnp.int32) for _ in range(nchunk)],
            pltpu.VMEM((_CHUNK, d), jnp.float32),
            [pltpu.VMEM((_CHUNK, d), jnp.float32) for _ in range(_NBUF)],
            [pltpu.SemaphoreType.DMA for _ in range(_NBUF)],
            [pltpu.SemaphoreType.DMA for _ in range(_NBUF)],
            pltpu.SemaphoreType.DMA,
            pltpu.SemaphoreType.DMA,
        ],
    )
    def body(x_ref, pe_ref, tab_ref, out_ref, idx_v, pe_v, toks, gsem, osem,
             psem, ixsem):
        wid = lax.axis_index("s") * _NC + lax.axis_index("c")
        p0 = wid * _POS_PER_W
        pec = pltpu.async_copy(pe_ref.at[pl.ds(p0, _CHUNK)], pe_v, psem)
        ixc = []
        for c in range(nchunk):
            h, b = divmod(c, batch)
            ixc.append(pltpu.async_copy(
                x_ref.at[b, pl.ds(p0 + h * _CHUNK, _CHUNK)], idx_v[c], ixsem
            ))
        for h_ in ixc:
            h_.wait()

        def gather(c):
            buf = c % _NBUF
            return pltpu.async_copy(tab_ref.at[idx_v[c]], toks[buf], gsem[buf])

        def fire_out(c):
            h, b = divmod(c, batch)
            buf = c % _NBUF
            return pltpu.async_copy(
                toks[buf],
                out_ref.at[pl.ds(b * seq_len + p0 + h * _CHUNK, _CHUNK)],
                osem[buf],
            )

        def compute_rows(bufA, bufB, r0, r1):
            def row(r, carry):
                for j in range(nv2):
                    sa = pl.ds(j * 32, 16)
                    sb = pl.ds(j * 32 + 16, 16)
                    pa = pe_v[r, sa]
                    pb = pe_v[r, sb]
                    bufA[r, sa] = bufA[r, sa] * scale + pa
                    bufB[r, sa] = bufB[r, sa] * scale + pa
                    bufA[r, sb] = bufA[r, sb] * scale + pb
                    bufB[r, sb] = bufB[r, sb] * scale + pb
                return carry
            lax.fori_loop(r0, r1, row, 0)

        ga = [None] * nchunk
        oc = [None] * nchunk
        for c in range(_NBUF - 1):
            ga[c] = gather(c)
        pec.wait()

        pairs_per_h = npair // halves
        for p in range(npair):
            cA, cB = 2 * p, 2 * p + 1
            bufA, bufB = toks[cA % _NBUF], toks[cB % _NBUF]
            ga[cA].wait()
            ga[cB].wait()
            if p == pairs_per_h:
                pec.wait()
            if p == 0:
                ga[3] = gather(3)
                compute_rows(bufA, bufB, 0, _CHUNK)
            else:
                compute_rows(bufA, bufB, 0, half_rows)
                for nc_ in (2 * p + 2, 2 * p + 3):
                    if nc_ < nchunk:
                        prev = nc_ - _NBUF
                        oc[prev].wait()
                        ga[nc_] = gather(nc_)
                compute_rows(bufA, bufB, half_rows, _CHUNK)
            if p == pairs_per_h - 1:
                pec = pltpu.async_copy(
                    pe_ref.at[pl.ds(p0 + _CHUNK, _CHUNK)], pe_v, psem
                )
            oc[cA] = fire_out(cA)
            oc[cB] = fire_out(cB)

        for c in range(nchunk - _NBUF, nchunk):
            oc[c].wait()

    return body(x, pe, table)


def kernel(x, table):
    batch, seq_len = x.shape
    d = table.shape[1]
    pe = jnp.asarray(_PE[:seq_len])
    out = _embed(x.astype(jnp.int32), pe, table, batch, seq_len)
    return out.reshape(batch, seq_len, d)
